# Initial kernel scaffold; baseline (speedup 1.0000x reference)
import jax
import jax.numpy as jnp
from jax.experimental import pallas as pl


def kernel(X):
    raise NotImplementedError("write your pallas kernel here")


# speedup vs baseline: 59.5394x; 59.5394x over previous
import functools

import jax
import jax.numpy as jnp
from jax import lax
from jax.experimental import pallas as pl
from jax.experimental.pallas import tpu as pltpu
from jax.experimental.pallas import tpu_sc as plsc

BINS = 256
NB2 = BINS * BINS
H = 512
W = 512
NCH = 24
CHUNK = 32
NCHUNK = H // CHUNK
ROWW = W + 16
NVEC = W // 16

C0 = -0.5 * 3.141592653589793
C1 = 0.5 * 5.1677127800499700
C2 = -0.5 * 2.5501640398773455
C3 = 0.5 * 0.5992645293207921
C4 = -0.5 * 0.0821458866111282

CLIP_HI = 254.99998474121094


def _sc_body(x_hbm, out_hbm, hist, inbuf, w0a, w1a, a0a, sem0, sem1):
    info = plsc.get_sparse_core_info()
    nc = info.num_cores
    wid = lax.axis_index("s") * nc + lax.axis_index("c")
    ch = jnp.minimum(wid, NCH - 1)

    zf = jnp.zeros((16,), jnp.float32)

    def zero_body(i, _):
        hist[pl.ds(i * 16, 16)] = zf
        return 0

    lax.fori_loop(0, NB2 // 16, zero_body, 0)
    w0a[pl.ds(W, 16)] = zf
    w1a[pl.ds(W, 16)] = zf
    a0a[pl.ds(W, 16)] = jnp.zeros((16,), jnp.int32)

    def row_body(buf):
        def body(r, _):
            def px_body(v, _):
                x = inbuf[buf, r, pl.ds(v * 16, 16)]
                x = jnp.minimum(jnp.maximum(x, 0.0), CLIP_HI)
                k = x.astype(jnp.int32)
                f = x - k.astype(jnp.float32)
                u = f - 0.5
                z = u * u
                p = C3 + z * C4
                p = C2 + z * p
                p = C1 + z * p
                p = C0 + z * p
                w0 = 0.5 + u * p
                w0a[pl.ds(v * 16, 16)] = w0
                w1a[pl.ds(v * 16, 16)] = 1.0 - w0
                a0a[pl.ds(v * 16, 16)] = k * BINS
                return 0

            lax.fori_loop(0, NVEC, px_body, 0)

            def pair_body(v, _):
                base = v * 16
                w0l = w0a[pl.ds(base, 16)]
                w1l = w1a[pl.ds(base, 16)]
                a0l = a0a[pl.ds(base, 16)]
                w0r = w0a[pl.ds(base + 1, 16)]
                w1r = w1a[pl.ds(base + 1, 16)]
                a0r = a0a[pl.ds(base + 1, 16)]
                k0r = lax.shift_right_logical(a0r, 8)
                f00 = a0l + k0r
                plsc.addupdate_scatter(hist, [f00], w0l * w0r)
                plsc.addupdate_scatter(hist, [f00 + 1], w0l * w1r)
                plsc.addupdate_scatter(hist, [f00 + BINS], w1l * w0r)
                plsc.addupdate_scatter(hist, [f00 + (BINS + 1)], w1l * w1r)
                return 0

            lax.fori_loop(0, NVEC, pair_body, 0)
            return 0

        lax.fori_loop(0, CHUNK, body, 0)

    sems = (sem0, sem1)
    copies = [None] * NCHUNK
    copies[0] = pltpu.async_copy(x_hbm.at[ch, pl.ds(0, CHUNK)], inbuf.at[0], sem0)
    for g in range(NCHUNK):
        if g + 1 < NCHUNK:
            copies[g + 1] = pltpu.async_copy(
                x_hbm.at[ch, pl.ds((g + 1) * CHUNK, CHUNK)],
                inbuf.at[(g + 1) % 2],
                sems[(g + 1) % 2],
            )
        copies[g].wait()
        row_body(g % 2)

    def max_body(i, acc):
        return jnp.maximum(acc, hist[pl.ds(i * 16, 16)])

    acc = lax.fori_loop(0, NB2 // 16, max_body, zf)
    inv = 1.0 / jnp.broadcast_to(jnp.max(acc), (16,))

    def scale_body(i, _):
        hist[pl.ds(i * 16, 16)] = hist[pl.ds(i * 16, 16)] * inv
        return 0

    lax.fori_loop(0, NB2 // 16, scale_body, 0)

    @pl.when(wid < NCH)
    def _():
        pltpu.sync_copy(hist, out_hbm.at[ch])


def kernel(X):
    B, C, h, w = X.shape
    x = X.reshape(B * C, h, w)
    mesh = plsc.VectorSubcoreMesh(core_axis_name="c", subcore_axis_name="s")
    hist_fn = pl.kernel(
        _sc_body,
        out_type=jax.ShapeDtypeStruct((NCH, NB2), jnp.float32),
        mesh=mesh,
        compiler_params=pltpu.CompilerParams(needs_layout_passes=False),
        scratch_types=[
            pltpu.VMEM((NB2,), jnp.float32),
            pltpu.VMEM((2, CHUNK, W), jnp.float32),
            pltpu.VMEM((ROWW,), jnp.float32),
            pltpu.VMEM((ROWW,), jnp.float32),
            pltpu.VMEM((ROWW,), jnp.int32),
            pltpu.SemaphoreType.DMA,
            pltpu.SemaphoreType.DMA,
        ],
    )
    out = hist_fn(x)
    return out.reshape(B, C, BINS, BINS)


# speedup vs baseline: 67.8040x; 1.1388x over previous
import functools

import jax
import jax.numpy as jnp
from jax import lax
from jax.experimental import pallas as pl
from jax.experimental.pallas import tpu as pltpu
from jax.experimental.pallas import tpu_sc as plsc

BINS = 256
NB2 = BINS * BINS
H = 512
W = 512
NCH = 24
CHUNK = 32
NCHUNK = H // CHUNK
ROWW = W + 16
NVEC = W // 16

C0 = -0.5 * 3.141592653589793
C1 = 0.5 * 5.1677127800499700
C2 = -0.5 * 2.5501640398773455
C3 = 0.5 * 0.5992645293207921
C4 = -0.5 * 0.0821458866111282

CLIP_HI = 254.99998474121094


def _sc_body(x_hbm, out_hbm, hist, inbuf, w0a, a0a, sem0, sem1):
    info = plsc.get_sparse_core_info()
    nc = info.num_cores
    wid = lax.axis_index("s") * nc + lax.axis_index("c")
    ch = jnp.minimum(wid, NCH - 1)

    zf = jnp.zeros((16,), jnp.float32)

    def zero_body(i, _):
        hist[pl.ds(i * 16, 16)] = zf
        return 0

    lax.fori_loop(0, NB2 // 16, zero_body, 0)
    w0a[pl.ds(W, 16)] = zf
    a0a[pl.ds(W, 16)] = jnp.zeros((16,), jnp.int32)

    def _pixel(x):
        x = jnp.minimum(jnp.maximum(x, 0.0), CLIP_HI)
        k = x.astype(jnp.int32)
        f = x - k.astype(jnp.float32)
        u = f - 0.5
        z = u * u
        z2 = z * z
        pa = C0 + z * C1
        pb = C2 + z * C3
        p = pa + z2 * (pb + z2 * C4)
        w0 = 0.5 + u * p
        return w0, 1.0 - w0, k * BINS

    def _store_px(base, w0, a0):
        w0a[pl.ds(base, 16)] = w0
        a0a[pl.ds(base, 16)] = a0

    def _pair(w0l, w1l, a0l, rbase):
        w0r = w0a[pl.ds(rbase, 16)]
        a0r = a0a[pl.ds(rbase, 16)]
        w1r = 1.0 - w0r
        f00 = a0l + lax.shift_right_logical(a0r, 8)
        plsc.addupdate_scatter(hist, [f00], w0l * w0r)
        plsc.addupdate_scatter(hist, [f00 + 1], w0l * w1r)
        plsc.addupdate_scatter(hist, [f00 + BINS], w1l * w0r)
        plsc.addupdate_scatter(hist, [f00 + (BINS + 1)], w1l * w1r)

    def row_body(buf):
        def body(r, _):
            p0 = _pixel(inbuf[buf, r, pl.ds(0, 16)])
            _store_px(0, p0[0], p0[2])
            p1 = _pixel(inbuf[buf, r, pl.ds(16, 16)])
            _store_px(16, p1[0], p1[2])
            _pair(p0[0], p0[1], p0[2], 1)

            def ploop(k, carry):
                base = k * 32
                pA = _pixel(inbuf[buf, r, pl.ds(base, 16)])
                _store_px(base, pA[0], pA[2])
                pB = _pixel(inbuf[buf, r, pl.ds(base + 16, 16)])
                _store_px(base + 16, pB[0], pB[2])
                _pair(carry[0], carry[1], carry[2], base - 15)
                _pair(pA[0], pA[1], pA[2], base + 1)
                return pB

            pz = lax.fori_loop(1, NVEC // 2, ploop, p1)
            _pair(pz[0], pz[1], pz[2], W - 15)
            return 0

        lax.fori_loop(0, CHUNK, body, 0)

    sems = (sem0, sem1)
    copies = [None] * NCHUNK
    copies[0] = pltpu.async_copy(x_hbm.at[ch, pl.ds(0, CHUNK)], inbuf.at[0], sem0)
    for g in range(NCHUNK):
        if g + 1 < NCHUNK:
            copies[g + 1] = pltpu.async_copy(
                x_hbm.at[ch, pl.ds((g + 1) * CHUNK, CHUNK)],
                inbuf.at[(g + 1) % 2],
                sems[(g + 1) % 2],
            )
        copies[g].wait()
        row_body(g % 2)

    def max_body(i, acc):
        return jnp.maximum(acc, hist[pl.ds(i * 16, 16)])

    acc = lax.fori_loop(0, NB2 // 16, max_body, zf)
    inv = 1.0 / jnp.broadcast_to(jnp.max(acc), (16,))

    def scale_body(i, _):
        hist[pl.ds(i * 16, 16)] = hist[pl.ds(i * 16, 16)] * inv
        return 0

    lax.fori_loop(0, NB2 // 16, scale_body, 0)

    @pl.when(wid < NCH)
    def _():
        pltpu.sync_copy(hist, out_hbm.at[ch])


def kernel(X):
    B, C, h, w = X.shape
    x = X.reshape(B * C, h, w)
    mesh = plsc.VectorSubcoreMesh(core_axis_name="c", subcore_axis_name="s")
    hist_fn = pl.kernel(
        _sc_body,
        out_type=jax.ShapeDtypeStruct((NCH, NB2), jnp.float32),
        mesh=mesh,
        compiler_params=pltpu.CompilerParams(needs_layout_passes=False),
        scratch_types=[
            pltpu.VMEM((NB2,), jnp.float32),
            pltpu.VMEM((2, CHUNK, W), jnp.float32),
            pltpu.VMEM((ROWW,), jnp.float32),
            pltpu.VMEM((ROWW,), jnp.int32),
            pltpu.SemaphoreType.DMA,
            pltpu.SemaphoreType.DMA,
        ],
    )
    out = hist_fn(x)
    return out.reshape(B, C, BINS, BINS)


# speedup vs baseline: 86.2157x; 1.2715x over previous
import functools

import jax
import jax.numpy as jnp
from jax import lax
from jax.experimental import pallas as pl
from jax.experimental.pallas import tpu as pltpu
from jax.experimental.pallas import tpu_sc as plsc

BINS = 256
NB2 = BINS * BINS
H = 512
W = 512
NCH = 24
CHUNK = 32
NCHUNK = H // CHUNK
ROWW = W + 16
NVEC = W // 16

C0 = -0.5 * 3.141592653589793
C1 = 0.5 * 5.1677127800499700
C2 = -0.5 * 2.5501640398773455
C3 = 0.5 * 0.5992645293207921
C4 = -0.5 * 0.0821458866111282

CLIP_HI = 254.99998474121094


def _sc_body(x_hbm, out_hbm, hist, inbuf, w0a, a0a, sem0, sem1):
    info = plsc.get_sparse_core_info()
    nc = info.num_cores
    wid = lax.axis_index("s") * nc + lax.axis_index("c")
    ch = jnp.minimum(wid, NCH - 1)

    zf = jnp.zeros((16,), jnp.float32)

    def zero_body(i, _):
        hist[pl.ds(i * 16, 16)] = zf
        return 0

    lax.fori_loop(0, NB2 // 16, zero_body, 0)
    w0a[pl.ds(W, 16)] = zf
    a0a[pl.ds(W, 16)] = jnp.zeros((16,), jnp.int32)
    mask15 = lax.iota(jnp.int32, 16) < (16 - 1)

    def _pixel(x):
        x = jnp.minimum(jnp.maximum(x, 0.0), CLIP_HI)
        k = x.astype(jnp.int32)
        f = x - k.astype(jnp.float32)
        u = f - 0.5
        z = u * u
        z2 = z * z
        pa = C0 + z * C1
        pb = C2 + z * C3
        p = pa + z2 * (pb + z2 * C4)
        w0 = 0.5 + u * p
        return w0, 1.0 - w0, k * BINS

    def _store_px(base, w0, a0):
        w0a[pl.ds(base, 16)] = w0
        a0a[pl.ds(base, 16)] = a0

    def _pair(w0l, w1l, a0l, rbase):
        w0r = w0a[pl.ds(rbase, 16)]
        a0r = a0a[pl.ds(rbase, 16)]
        w1r = 1.0 - w0r
        f00 = a0l + lax.shift_right_logical(a0r, 8)
        plsc.addupdate_scatter(hist, [f00], w0l * w0r)
        plsc.addupdate_scatter(hist, [f00 + 1], w0l * w1r)
        plsc.addupdate_scatter(hist, [f00 + BINS], w1l * w0r)
        plsc.addupdate_scatter(hist, [f00 + (BINS + 1)], w1l * w1r)

    def row_body(buf):
        def body(r, _):
            def px_body(k, _):
                base = k * 32
                pA = _pixel(inbuf[buf, r, pl.ds(base, 16)])
                pB = _pixel(inbuf[buf, r, pl.ds(base + 16, 16)])
                _store_px(base, pA[0], pA[2])
                _store_px(base + 16, pB[0], pB[2])
                return 0

            lax.fori_loop(0, NVEC // 2, px_body, 0)

            def pair_body(k, _):
                base = k * 32
                w0lA = w0a[pl.ds(base, 16)]
                a0lA = a0a[pl.ds(base, 16)]
                w0lB = w0a[pl.ds(base + 16, 16)]
                a0lB = a0a[pl.ds(base + 16, 16)]
                _pair(w0lA, 1.0 - w0lA, a0lA, base + 1)
                _pair(w0lB, 1.0 - w0lB, a0lB, base + 17)
                return 0

            lax.fori_loop(0, NVEC // 2 - 1, pair_body, 0)

            base = (NVEC - 2) * 16
            w0lA = w0a[pl.ds(base, 16)]
            a0lA = a0a[pl.ds(base, 16)]
            _pair(w0lA, 1.0 - w0lA, a0lA, base + 1)
            w0lB = w0a[pl.ds(base + 16, 16)]
            a0lB = a0a[pl.ds(base + 16, 16)]
            w0r = w0a[pl.ds(base + 17, 16)]
            a0r = a0a[pl.ds(base + 17, 16)]
            w1lB = 1.0 - w0lB
            w1r = 1.0 - w0r
            f00 = a0lB + lax.shift_right_logical(a0r, 8)
            plsc.addupdate_scatter(hist, [f00], w0lB * w0r)
            plsc.addupdate_scatter(hist, [f00 + 1], w0lB * w1r, mask=mask15)
            plsc.addupdate_scatter(hist, [f00 + BINS], w1lB * w0r)
            plsc.addupdate_scatter(hist, [f00 + (BINS + 1)], w1lB * w1r, mask=mask15)
            return 0

        lax.fori_loop(0, CHUNK, body, 0)

    sems = (sem0, sem1)
    copies = [None] * NCHUNK
    copies[0] = pltpu.async_copy(x_hbm.at[ch, pl.ds(0, CHUNK)], inbuf.at[0], sem0)
    for g in range(NCHUNK):
        if g + 1 < NCHUNK:
            copies[g + 1] = pltpu.async_copy(
                x_hbm.at[ch, pl.ds((g + 1) * CHUNK, CHUNK)],
                inbuf.at[(g + 1) % 2],
                sems[(g + 1) % 2],
            )
        copies[g].wait()
        row_body(g % 2)

    def max_body(i, acc):
        return jnp.maximum(acc, hist[pl.ds(i * 16, 16)])

    acc = lax.fori_loop(0, NB2 // 16, max_body, zf)
    inv = 1.0 / jnp.broadcast_to(jnp.max(acc), (16,))

    def scale_body(i, _):
        hist[pl.ds(i * 16, 16)] = hist[pl.ds(i * 16, 16)] * inv
        return 0

    lax.fori_loop(0, NB2 // 16, scale_body, 0)

    @pl.when(wid < NCH)
    def _():
        pltpu.sync_copy(hist, out_hbm.at[ch])


def kernel(X):
    B, C, h, w = X.shape
    x = X.reshape(B * C, h, w)
    mesh = plsc.VectorSubcoreMesh(core_axis_name="c", subcore_axis_name="s")
    hist_fn = pl.kernel(
        _sc_body,
        out_type=jax.ShapeDtypeStruct((NCH, NB2), jnp.float32),
        mesh=mesh,
        compiler_params=pltpu.CompilerParams(needs_layout_passes=False),
        scratch_types=[
            pltpu.VMEM((NB2,), jnp.float32),
            pltpu.VMEM((2, CHUNK, W), jnp.float32),
            pltpu.VMEM((ROWW,), jnp.float32),
            pltpu.VMEM((ROWW,), jnp.int32),
            pltpu.SemaphoreType.DMA,
            pltpu.SemaphoreType.DMA,
        ],
    )
    out = hist_fn(x)
    return out.reshape(B, C, BINS, BINS)
